# Initial kernel scaffold; baseline (speedup 1.0000x reference)
import jax
import jax.numpy as jnp
from jax.experimental import pallas as pl


def kernel(user, movie, user_table, movie_table, W1, b1, W2, b2):
    raise NotImplementedError("write your pallas kernel here")


# speedup vs baseline: 1.1427x; 1.1427x over previous
import functools

import jax
import jax.numpy as jnp
from jax import lax
from jax.experimental import pallas as pl
from jax.experimental.pallas import tpu as pltpu
from jax.experimental.pallas import tpu_sc as plsc

BATCH = 16384
NF = 64
H1 = 512

NUM_WORKERS = 32
BPW = BATCH // NUM_WORKERS
CHUNK = 128
NCHUNK = BPW // CHUNK


def _gather_body(uidx_hbm, midx_hbm, utab_hbm, mtab_hbm, ue_out, me_out,
                 uidx_v, midx_v, urows_v, mrows_v, sem):
    wid = lax.axis_index("s") * 2 + lax.axis_index("c")
    row0 = wid * NCHUNK
    base = wid * BPW

    pltpu.sync_copy(uidx_hbm.at[pl.ds(row0, NCHUNK)], uidx_v)
    pltpu.sync_copy(midx_hbm.at[pl.ds(row0, NCHUNK)], midx_v)

    copies = []
    for j in range(NCHUNK):
        c = pltpu.make_async_copy(
            utab_hbm.at[uidx_v.at[j]], urows_v.at[pl.ds(j * CHUNK, CHUNK)], sem)
        c.start()
        copies.append(c)
        c = pltpu.make_async_copy(
            mtab_hbm.at[midx_v.at[j]], mrows_v.at[pl.ds(j * CHUNK, CHUNK)], sem)
        c.start()
        copies.append(c)
    for c in copies:
        c.wait()

    pltpu.sync_copy(urows_v, ue_out.at[pl.ds(base, BPW)])
    pltpu.sync_copy(mrows_v, me_out.at[pl.ds(base, BPW)])


_gather = functools.partial(
    pl.kernel,
    mesh=plsc.VectorSubcoreMesh(core_axis_name="c", subcore_axis_name="s"),
    out_type=[
        jax.ShapeDtypeStruct((BATCH, NF), jnp.float32),
        jax.ShapeDtypeStruct((BATCH, NF), jnp.float32),
    ],
    scratch_types=[
        pltpu.VMEM((NCHUNK, CHUNK), jnp.int32),
        pltpu.VMEM((NCHUNK, CHUNK), jnp.int32),
        pltpu.VMEM((BPW, NF), jnp.float32),
        pltpu.VMEM((BPW, NF), jnp.float32),
        pltpu.SemaphoreType.DMA,
    ],
    compiler_params=pltpu.CompilerParams(use_tc_tiling_on_sc=False),
)(_gather_body)


MLP_BLOCK = 1024


def _mlp_body(ue_ref, me_ref, w1u_ref, w1m_ref, b1_ref, w2t_ref, b2_ref, out_ref):
    h = (jnp.dot(ue_ref[...], w1u_ref[...], preferred_element_type=jnp.float32)
         + jnp.dot(me_ref[...], w1m_ref[...], preferred_element_type=jnp.float32)
         + b1_ref[...])
    h = jnp.maximum(h, 0.0)
    out_ref[...] = jnp.sum(h * w2t_ref[...], axis=1, keepdims=True) + b2_ref[...]


_mlp = pl.pallas_call(
    _mlp_body,
    grid=(BATCH // MLP_BLOCK,),
    in_specs=[
        pl.BlockSpec((MLP_BLOCK, NF), lambda i: (i, 0)),
        pl.BlockSpec((MLP_BLOCK, NF), lambda i: (i, 0)),
        pl.BlockSpec((NF, H1), lambda i: (0, 0)),
        pl.BlockSpec((NF, H1), lambda i: (0, 0)),
        pl.BlockSpec((1, H1), lambda i: (0, 0)),
        pl.BlockSpec((1, H1), lambda i: (0, 0)),
        pl.BlockSpec((1, 1), lambda i: (0, 0)),
    ],
    out_specs=pl.BlockSpec((MLP_BLOCK, 1), lambda i: (i, 0)),
    out_shape=jax.ShapeDtypeStruct((BATCH, 1), jnp.float32),
    compiler_params=pltpu.CompilerParams(
        dimension_semantics=("arbitrary",),
    ),
)


def kernel(user, movie, user_table, movie_table, W1, b1, W2, b2):
    uidx = user.astype(jnp.int32).reshape(BATCH // CHUNK, CHUNK)
    midx = movie.astype(jnp.int32).reshape(BATCH // CHUNK, CHUNK)
    ue, me = _gather(uidx, midx, user_table, movie_table)
    w1u = W1[:NF]
    w1m = W1[NF:]
    return _mlp(ue, me, w1u, w1m, b1.reshape(1, H1), W2.reshape(1, H1), b2.reshape(1, 1))


# speedup vs baseline: 1.2743x; 1.1151x over previous
import functools

import jax
import jax.numpy as jnp
from jax import lax
from jax.experimental import pallas as pl
from jax.experimental.pallas import tpu as pltpu
from jax.experimental.pallas import tpu_sc as plsc

BATCH = 16384
NF = 64
H1 = 512

NUM_WORKERS = 32
BPW = BATCH // NUM_WORKERS
CHUNK = 128
NCHUNK = BPW // CHUNK


def _gather_body(uidx_hbm, midx_hbm, utab_hbm, mtab_hbm, x_out,
                 uidx_v, midx_v, urows_v, mrows_v, sem):
    wid = lax.axis_index("s") * 2 + lax.axis_index("c")
    row0 = wid * NCHUNK
    base = wid * BPW

    pltpu.sync_copy(uidx_hbm.at[pl.ds(row0, NCHUNK)], uidx_v)
    pltpu.sync_copy(midx_hbm.at[pl.ds(row0, NCHUNK)], midx_v)

    copies = []
    for j in range(NCHUNK):
        c = pltpu.make_async_copy(
            utab_hbm.at[uidx_v.at[j]], urows_v.at[pl.ds(j * CHUNK, CHUNK)], sem)
        c.start()
        copies.append(c)
        c = pltpu.make_async_copy(
            mtab_hbm.at[midx_v.at[j]], mrows_v.at[pl.ds(j * CHUNK, CHUNK)], sem)
        c.start()
        copies.append(c)
    for c in copies:
        c.wait()

    pltpu.sync_copy(urows_v, x_out.at[pl.ds(base, BPW), pl.ds(0, NF)])
    pltpu.sync_copy(mrows_v, x_out.at[pl.ds(base, BPW), pl.ds(NF, NF)])


_gather = functools.partial(
    pl.kernel,
    mesh=plsc.VectorSubcoreMesh(core_axis_name="c", subcore_axis_name="s"),
    out_type=jax.ShapeDtypeStruct((BATCH, 2 * NF), jnp.float32),
    scratch_types=[
        pltpu.VMEM((NCHUNK, CHUNK), jnp.int32),
        pltpu.VMEM((NCHUNK, CHUNK), jnp.int32),
        pltpu.VMEM((BPW, NF), jnp.float32),
        pltpu.VMEM((BPW, NF), jnp.float32),
        pltpu.SemaphoreType.DMA,
    ],
    compiler_params=pltpu.CompilerParams(use_tc_tiling_on_sc=False),
)(_gather_body)


MLP_BLOCK = 1024


def _mlp_body(x_ref, w1_ref, b1_ref, w2t_ref, b2_ref, out_ref):
    h = jnp.dot(x_ref[...], w1_ref[...], preferred_element_type=jnp.float32) + b1_ref[...]
    h = jnp.maximum(h, 0.0)
    out_ref[...] = jnp.sum(h * w2t_ref[...], axis=1, keepdims=True) + b2_ref[...]


_mlp = pl.pallas_call(
    _mlp_body,
    grid=(BATCH // MLP_BLOCK,),
    in_specs=[
        pl.BlockSpec((MLP_BLOCK, 2 * NF), lambda i: (i, 0)),
        pl.BlockSpec((2 * NF, H1), lambda i: (0, 0)),
        pl.BlockSpec((1, H1), lambda i: (0, 0)),
        pl.BlockSpec((1, H1), lambda i: (0, 0)),
        pl.BlockSpec((1, 1), lambda i: (0, 0)),
    ],
    out_specs=pl.BlockSpec((MLP_BLOCK, 1), lambda i: (i, 0)),
    out_shape=jax.ShapeDtypeStruct((BATCH, 1), jnp.float32),
    compiler_params=pltpu.CompilerParams(
        dimension_semantics=("arbitrary",),
    ),
)


def kernel(user, movie, user_table, movie_table, W1, b1, W2, b2):
    uidx = user.astype(jnp.int32).reshape(BATCH // CHUNK, CHUNK)
    midx = movie.astype(jnp.int32).reshape(BATCH // CHUNK, CHUNK)
    x = _gather(uidx, midx, user_table, movie_table)
    return _mlp(x, W1, b1.reshape(1, H1), W2.reshape(1, H1), b2.reshape(1, 1))
